# Initial kernel scaffold; baseline (speedup 1.0000x reference)
import jax
import jax.numpy as jnp
from jax.experimental import pallas as pl


def kernel(adj, embeds, batch_size):
    raise NotImplementedError("write your pallas kernel here")


# speedup vs baseline: 1.1868x; 1.1868x over previous
import jax
import jax.numpy as jnp
from jax.experimental import pallas as pl


def _gcn_matmul_kernel(adj_ref, emb_ref, out_ref):
    out_ref[...] = jnp.dot(
        adj_ref[...], emb_ref[...], preferred_element_type=jnp.float32
    )


def kernel(adj, embeds, batch_size):
    adj = adj.astype(jnp.float32)
    embeds = embeds.astype(jnp.float32)
    n, k = adj.shape
    d = embeds.shape[1]
    bm = 512
    return pl.pallas_call(
        _gcn_matmul_kernel,
        grid=(n // bm,),
        in_specs=[
            pl.BlockSpec((bm, k), lambda i: (i, 0)),
            pl.BlockSpec((k, d), lambda i: (0, 0)),
        ],
        out_specs=pl.BlockSpec((bm, d), lambda i: (i, 0)),
        out_shape=jax.ShapeDtypeStruct((n, d), jnp.float32),
    )(adj, embeds)
